# baseline (speedup 1.0000x reference)
import jax
import jax.numpy as jnp
from jax import lax
from jax.experimental import pallas as pl
from jax.experimental.pallas import tpu as pltpu
from jax.experimental.pallas import tpu_sc as plsc

_N = 10000
_E = 320000
_D = 128
_F = 16
_H = 8
_B = 16384
_HF = _H * _F
_W144 = _HF + 16

_NC = 2
_NS = 16
_NW = _NC * _NS
_EPT = _E // _NW
_CH = 100
_NCH = _EPT // _CH
_BLK = 1000
_CPB = _BLK // _CH
_NBLK = _EPT // _BLK
_RA = 632
_BPT = _B // _NW

_mesh = plsc.VectorSubcoreMesh(
    core_axis_name="c", subcore_axis_name="s",
    num_cores=_NC, num_subcores=_NS)


def _tc_proj_body(x_ref, wf_ref, asv_ref, adv_ref, zs_ref, dp_ref):
    z2 = jnp.dot(x_ref[...], wf_ref[...], preferred_element_type=jnp.float32)
    col = lax.broadcasted_iota(jnp.int32, (_HF, 16), 1)
    rowh = lax.broadcasted_iota(jnp.int32, (_HF, 16), 0) // _F
    sel = col == rowh
    a_s = jnp.where(sel, asv_ref[...], 0.0)
    a_d = jnp.where(sel, adv_ref[...], 0.0)
    sp = jnp.dot(z2, a_s, preferred_element_type=jnp.float32)
    zs_ref[...] = jnp.concatenate([z2, sp], axis=1)
    dp_ref[...] = jnp.dot(z2, a_d, preferred_element_type=jnp.float32)


_tc_proj = pl.pallas_call(
    _tc_proj_body,
    out_shape=[
        jax.ShapeDtypeStruct((_N, _W144), jnp.float32),
        jax.ShapeDtypeStruct((_N, 16), jnp.float32),
    ],
)


def _sc_edge_body(src2_hbm, dst2_hbm, zs_hbm, dp_hbm, acc_hbm,
                  sblk0, sblk1, dblk0, dblk1, g0, g1, db0, db1, acc,
                  sem_blk, sem_g0, sem_g1, sem_s0, sem_s1):
    cid = lax.axis_index("c")
    sid = lax.axis_index("s")
    wid = sid * _NC + cid
    z16 = jnp.zeros((16,), jnp.float32)
    sblk = (sblk0, sblk1)
    dblk = (dblk0, dblk1)
    g = (g0, g1)
    db = (db0, db1)
    sem_g = (sem_g0, sem_g1)
    sem_s = (sem_s0, sem_s1)

    def zero_body(r, carry):
        for j in range(_W144 // 16):
            g0[r, pl.ds(j * 16, 16)] = z16
        return carry
    lax.fori_loop(0, _CH, zero_body, 0)
    rbase = sid * _RA

    def init_rows(nrows):
        done = 0
        while done < nrows:
            n = min(_CH, nrows - done)
            pltpu.sync_copy(g0.at[pl.ds(0, n)],
                            acc.at[pl.ds(rbase + done, n)])
            done += n

    @pl.when(sid < _NS - 1)
    def _():
        init_rows(_RA)

    @pl.when(sid == _NS - 1)
    def _():
        init_rows(_N - (_NS - 1) * _RA)
    plsc.subcore_barrier()

    rowbase = wid * (_EPT // _CH)

    def start_blk(blk_idx, slot):
        r0 = rowbase + blk_idx * _CPB
        pltpu.async_copy(src2_hbm.at[pl.ds(r0, _CPB)], sblk[slot], sem_blk)
        pltpu.async_copy(dst2_hbm.at[pl.ds(r0, _CPB)], dblk[slot], sem_blk)

    def wait_blk(slot):
        pltpu.make_async_copy(src2_hbm.at[pl.ds(0, _CPB)], sblk[slot],
                              sem_blk).wait()
        pltpu.make_async_copy(dst2_hbm.at[pl.ds(0, _CPB)], dblk[slot],
                              sem_blk).wait()

    def start_chunk(cib, slot, b):
        pltpu.async_copy(zs_hbm.at[sblk[slot].at[cib]], g[b], sem_g[b])
        pltpu.async_copy(dp_hbm.at[dblk[slot].at[cib]], db[b], sem_g[b])

    def wait_gathers(b):
        pltpu.make_async_copy(zs_hbm.at[sblk[0].at[0]], g[b],
                              sem_g[b]).wait()
        pltpu.make_async_copy(dp_hbm.at[dblk[0].at[0]], db[b],
                              sem_g[b]).wait()

    def start_scatter(cib, slot, b):
        pltpu.async_copy(g[b], acc.at[dblk[slot].at[cib]], sem_s[b],
                         add=True)

    def wait_scatter(b):
        pltpu.make_async_copy(g[b], acc.at[dblk[0].at[0]], sem_s[b]).wait()

    def compute(b, lo, hi):
        gb, dbb = g[b], db[b]

        @plsc.parallel_loop(lo, hi, step=1, unroll=4)
        def _(c):
            t = gb[c, pl.ds(_HF, 16)] + dbb[c]
            e = jnp.where(t > 0, t, 0.2 * t)
            w8 = jnp.exp(e)
            gb[c, pl.ds(_HF, 16)] = w8
            for j in range(_H):
                gb[c, pl.ds(j * 16, 16)] = gb[c, pl.ds(j * 16, 16)] * w8[j]

    start_blk(0, 0)
    wait_blk(0)
    start_chunk(0, 0, 0)
    pltpu.async_copy(acc.at[pl.ds(0, _CH)], g1, sem_s1)

    def pair_iter(i, carry):
        for j in range(2 * _CPB):
            b = j % 2
            slot = j // _CPB
            cib = j % _CPB
            ch = 2 * _CPB * i + j
            wait_gathers(b)
            if j == 1:
                start_blk(2 * i + 1, 1)
            if j == _CPB + 1:
                @pl.when(i < _NBLK // 2 - 1)
                def _():
                    start_blk(2 * i + 2, 0)
            compute(b, 0, 48)
            nxt_j = j + 1
            if nxt_j < 2 * _CPB:
                wait_scatter(1 - b)
                if nxt_j == _CPB:
                    wait_blk(1)
                start_chunk(nxt_j % _CPB, nxt_j // _CPB, 1 - b)
            else:
                @pl.when(i < _NBLK // 2 - 1)
                def _():
                    wait_scatter(1 - b)
                    wait_blk(0)
                    start_chunk(0, 0, 1 - b)
            compute(b, 48, _CH)
            start_scatter(cib, slot, b)
        return carry
    lax.fori_loop(0, _NBLK // 2, pair_iter, 0)

    wait_scatter(0)
    wait_scatter(1)
    plsc.subcore_barrier()

    @pl.when(sid < _NS - 1)
    def _():
        pltpu.sync_copy(acc.at[pl.ds(rbase, _RA)],
                        acc_hbm.at[cid, pl.ds(rbase, _RA)])

    @pl.when(sid == _NS - 1)
    def _():
        pltpu.sync_copy(acc.at[pl.ds(rbase, _N - (_NS - 1) * _RA)],
                        acc_hbm.at[cid, pl.ds(rbase, _N - (_NS - 1) * _RA)])


_sc_edge = pl.kernel(
    _sc_edge_body,
    out_type=jax.ShapeDtypeStruct((_NC, _N, _W144), jnp.float32),
    mesh=_mesh,
    compiler_params=pltpu.CompilerParams(use_tc_tiling_on_sc=False),
    scratch_types=[
        pltpu.VMEM((_CPB, _CH), jnp.int32),
        pltpu.VMEM((_CPB, _CH), jnp.int32),
        pltpu.VMEM((_CPB, _CH), jnp.int32),
        pltpu.VMEM((_CPB, _CH), jnp.int32),
        pltpu.VMEM((_CH, _W144), jnp.float32),
        pltpu.VMEM((_CH, _W144), jnp.float32),
        pltpu.VMEM((_CH, 16), jnp.float32),
        pltpu.VMEM((_CH, 16), jnp.float32),
        pltpu.VMEM_SHARED((_N, _W144), jnp.float32),
        pltpu.SemaphoreType.DMA,
        pltpu.SemaphoreType.DMA,
        pltpu.SemaphoreType.DMA,
        pltpu.SemaphoreType.DMA,
        pltpu.SemaphoreType.DMA,
    ],
)


def _tc_out_body(acc_ref, wp2_ref, bp_ref, uv_ref):
    both = acc_ref[0] + acc_ref[1]
    num = both[:, :_HF]
    den16 = both[:, _HF:]
    rowh = lax.broadcasted_iota(jnp.int32, (16, _HF), 0)
    colh = lax.broadcasted_iota(jnp.int32, (16, _HF), 1) // _F
    expand = jnp.where(rowh == colh, 1.0, 0.0)
    den128 = jnp.dot(den16, expand, preferred_element_type=jnp.float32)
    h = num / (den128 + 1e-9)
    hf = jnp.where(h > 0, h, jnp.exp(jnp.minimum(h, 0.0)) - 1.0)
    uv = jnp.dot(hf, wp2_ref[...], preferred_element_type=jnp.float32)
    colv = lax.broadcasted_iota(jnp.int32, (_N, 2), 1)
    uv_ref[...] = uv + jnp.where(colv == 0, bp_ref[0], 0.0)


_tc_out = pl.pallas_call(
    _tc_out_body,
    in_specs=[
        pl.BlockSpec(memory_space=pltpu.VMEM),
        pl.BlockSpec(memory_space=pltpu.VMEM),
        pl.BlockSpec(memory_space=pltpu.SMEM),
    ],
    out_shape=jax.ShapeDtypeStruct((_N, 2), jnp.float32),
)


def _sc_pair_body(uv_hbm, dis_hbm, mir_hbm, out_hbm, uvv, dbuf, mbuf, rbuf):
    cid = lax.axis_index("c")
    sid = lax.axis_index("s")
    wid = sid * _NC + cid
    base = wid * _BPT
    pltpu.sync_copy(uv_hbm, uvv)
    pltpu.sync_copy(dis_hbm.at[pl.ds(base, _BPT)], dbuf)
    pltpu.sync_copy(mir_hbm.at[pl.ds(base, _BPT)], mbuf)
    zero16 = jnp.zeros((16,), jnp.int32)
    one16 = jnp.ones((16,), jnp.int32)

    def pair(c0, carry):
        idd = dbuf[pl.ds(c0 * 16, 16)]
        idm = mbuf[pl.ds(c0 * 16, 16)]
        u = plsc.load_gather(uvv, [idd, zero16])
        v = plsc.load_gather(uvv, [idm, one16])
        t = u + v
        rbuf[pl.ds(c0 * 16, 16)] = 1.0 / (1.0 + jnp.exp(-t))
        return carry
    lax.fori_loop(0, _BPT // 16, pair, 0)
    pltpu.sync_copy(rbuf, out_hbm.at[pl.ds(base, _BPT)])


_sc_pair = pl.kernel(
    _sc_pair_body,
    out_type=jax.ShapeDtypeStruct((_B,), jnp.float32),
    mesh=_mesh,
    compiler_params=pltpu.CompilerParams(
        use_tc_tiling_on_sc=False, needs_layout_passes=False),
    scratch_types=[
        pltpu.VMEM((_N, 2), jnp.float32),
        pltpu.VMEM((_BPT,), jnp.int32),
        pltpu.VMEM((_BPT,), jnp.int32),
        pltpu.VMEM((_BPT,), jnp.float32),
    ],
)


def kernel(x, edge_index, diseases, mirnas, W, a_src, a_dst, Wp, bp):
    src2 = edge_index[0].reshape(_E // _CH, _CH)
    dst2 = edge_index[1].reshape(_E // _CH, _CH)
    wflat = jnp.transpose(W, (1, 0, 2)).reshape(_D, _HF)
    asv = a_src.reshape(_HF, 1)
    adv = a_dst.reshape(_HF, 1)
    wp2 = jnp.stack([Wp[:_HF, 0], Wp[_HF:, 0]], axis=1)

    zs, dpad = _tc_proj(x, wflat, asv, adv)
    acc_parts = _sc_edge(src2, dst2, zs, dpad)
    uv = _tc_out(acc_parts, wp2, bp)
    scores = _sc_pair(uv, diseases, mirnas)
    return scores.reshape(_B, 1)


# speedup vs baseline: 1.1140x; 1.1140x over previous
import jax
import jax.numpy as jnp
from jax import lax
from jax.experimental import pallas as pl
from jax.experimental.pallas import tpu as pltpu
from jax.experimental.pallas import tpu_sc as plsc

_N = 10000
_E = 320000
_D = 128
_F = 16
_H = 8
_B = 16384
_HF = _H * _F
_W144 = _HF + 16

_NC = 2
_NS = 16
_NW = _NC * _NS
_EPT = _E // _NW
_CH = 100
_NCH = _EPT // _CH
_BLK = 1000
_CPB = _BLK // _CH
_NBLK = _EPT // _BLK
_RA = 632
_BPT = _B // _NW

_mesh = plsc.VectorSubcoreMesh(
    core_axis_name="c", subcore_axis_name="s",
    num_cores=_NC, num_subcores=_NS)


def _tc_proj_body(x_ref, wf_ref, asv_ref, adv_ref, zs_ref, dp_ref):
    z2 = jnp.dot(x_ref[...], wf_ref[...], preferred_element_type=jnp.float32)
    col = lax.broadcasted_iota(jnp.int32, (_HF, 16), 1)
    rowh = lax.broadcasted_iota(jnp.int32, (_HF, 16), 0) // _F
    sel = col == rowh
    a_s = jnp.where(sel, asv_ref[...], 0.0)
    a_d = jnp.where(sel, adv_ref[...], 0.0)
    sp = jnp.dot(z2, a_s, preferred_element_type=jnp.float32)
    zs_ref[...] = jnp.concatenate([z2, sp], axis=1)
    dp_ref[...] = jnp.dot(z2, a_d, preferred_element_type=jnp.float32)


_tc_proj = pl.pallas_call(
    _tc_proj_body,
    out_shape=[
        jax.ShapeDtypeStruct((_N, _W144), jnp.float32),
        jax.ShapeDtypeStruct((_N, 16), jnp.float32),
    ],
)


def _sc_edge_body(src2_hbm, dst2_hbm, zs_hbm, dp_hbm, acc_hbm,
                  sblk0, sblk1, dblk0, dblk1, g0, g1, db0, db1, acc,
                  sem_blk, sem_g0, sem_g1, sem_s0, sem_s1):
    cid = lax.axis_index("c")
    sid = lax.axis_index("s")
    wid = sid * _NC + cid
    z16 = jnp.zeros((16,), jnp.float32)
    sblk = (sblk0, sblk1)
    dblk = (dblk0, dblk1)
    g = (g0, g1)
    db = (db0, db1)
    sem_g = (sem_g0, sem_g1)
    sem_s = (sem_s0, sem_s1)

    def zero_body(r, carry):
        for j in range(_W144 // 16):
            g0[r, pl.ds(j * 16, 16)] = z16
        return carry
    lax.fori_loop(0, _CH, zero_body, 0)
    rbase = sid * _RA

    def init_rows(nrows):
        done = 0
        while done < nrows:
            n = min(_CH, nrows - done)
            pltpu.sync_copy(g0.at[pl.ds(0, n)],
                            acc.at[pl.ds(rbase + done, n)])
            done += n

    @pl.when(sid < _NS - 1)
    def _():
        init_rows(_RA)

    @pl.when(sid == _NS - 1)
    def _():
        init_rows(_N - (_NS - 1) * _RA)
    plsc.subcore_barrier()

    rowbase = wid * (_EPT // _CH)

    def start_blk(blk_idx, slot):
        r0 = rowbase + blk_idx * _CPB
        pltpu.async_copy(src2_hbm.at[pl.ds(r0, _CPB)], sblk[slot], sem_blk)
        pltpu.async_copy(dst2_hbm.at[pl.ds(r0, _CPB)], dblk[slot], sem_blk)

    def wait_blk(slot):
        pltpu.make_async_copy(src2_hbm.at[pl.ds(0, _CPB)], sblk[slot],
                              sem_blk).wait()
        pltpu.make_async_copy(dst2_hbm.at[pl.ds(0, _CPB)], dblk[slot],
                              sem_blk).wait()

    def start_chunk(cib, slot, b):
        pltpu.async_copy(zs_hbm.at[sblk[slot].at[cib]], g[b], sem_g[b])
        pltpu.async_copy(dp_hbm.at[dblk[slot].at[cib]], db[b], sem_g[b])

    def wait_gathers(b):
        pltpu.make_async_copy(zs_hbm.at[sblk[0].at[0]], g[b],
                              sem_g[b]).wait()
        pltpu.make_async_copy(dp_hbm.at[dblk[0].at[0]], db[b],
                              sem_g[b]).wait()

    def start_scatter(cib, slot, b):
        pltpu.async_copy(g[b], acc.at[dblk[slot].at[cib]], sem_s[b],
                         add=True)

    def wait_scatter(b):
        pltpu.make_async_copy(g[b], acc.at[dblk[0].at[0]], sem_s[b]).wait()

    def compute(b, lo, hi):
        gb, dbb = g[b], db[b]

        @plsc.parallel_loop(lo, hi, step=1, unroll=4)
        def _(c):
            t = gb[c, pl.ds(_HF, 16)] + dbb[c]
            e = jnp.where(t > 0, t, 0.2 * t)
            w8 = jnp.exp(e)
            gb[c, pl.ds(_HF, 16)] = w8
            for j in range(_H):
                gb[c, pl.ds(j * 16, 16)] = gb[c, pl.ds(j * 16, 16)] * w8[j]

    start_blk(0, 0)
    wait_blk(0)
    start_chunk(0, 0, 0)
    pltpu.async_copy(acc.at[pl.ds(0, _CH)], g1, sem_s1)

    def pair_iter(i, carry):
        for j in range(2 * _CPB):
            b = j % 2
            slot = j // _CPB
            cib = j % _CPB
            ch = 2 * _CPB * i + j
            wait_gathers(b)
            if j == 1:
                start_blk(2 * i + 1, 1)
            if j == _CPB + 1:
                @pl.when(i < _NBLK // 2 - 1)
                def _():
                    start_blk(2 * i + 2, 0)
            nxt_j = j + 1
            if nxt_j < 2 * _CPB:
                wait_scatter(1 - b)
                if nxt_j == _CPB:
                    wait_blk(1)
                start_chunk(nxt_j % _CPB, nxt_j // _CPB, 1 - b)
            else:
                @pl.when(i < _NBLK // 2 - 1)
                def _():
                    wait_scatter(1 - b)
                    wait_blk(0)
                    start_chunk(0, 0, 1 - b)
            compute(b, 0, _CH)
            start_scatter(cib, slot, b)
        return carry
    lax.fori_loop(0, _NBLK // 2, pair_iter, 0)

    wait_scatter(0)
    wait_scatter(1)
    plsc.subcore_barrier()

    @pl.when(sid < _NS - 1)
    def _():
        pltpu.sync_copy(acc.at[pl.ds(rbase, _RA)],
                        acc_hbm.at[cid, pl.ds(rbase, _RA)])

    @pl.when(sid == _NS - 1)
    def _():
        pltpu.sync_copy(acc.at[pl.ds(rbase, _N - (_NS - 1) * _RA)],
                        acc_hbm.at[cid, pl.ds(rbase, _N - (_NS - 1) * _RA)])


_sc_edge = pl.kernel(
    _sc_edge_body,
    out_type=jax.ShapeDtypeStruct((_NC, _N, _W144), jnp.float32),
    mesh=_mesh,
    compiler_params=pltpu.CompilerParams(use_tc_tiling_on_sc=False),
    scratch_types=[
        pltpu.VMEM((_CPB, _CH), jnp.int32),
        pltpu.VMEM((_CPB, _CH), jnp.int32),
        pltpu.VMEM((_CPB, _CH), jnp.int32),
        pltpu.VMEM((_CPB, _CH), jnp.int32),
        pltpu.VMEM((_CH, _W144), jnp.float32),
        pltpu.VMEM((_CH, _W144), jnp.float32),
        pltpu.VMEM((_CH, 16), jnp.float32),
        pltpu.VMEM((_CH, 16), jnp.float32),
        pltpu.VMEM_SHARED((_N, _W144), jnp.float32),
        pltpu.SemaphoreType.DMA,
        pltpu.SemaphoreType.DMA,
        pltpu.SemaphoreType.DMA,
        pltpu.SemaphoreType.DMA,
        pltpu.SemaphoreType.DMA,
    ],
)


def _tc_out_body(acc_ref, wp2_ref, bp_ref, uv_ref):
    both = acc_ref[0] + acc_ref[1]
    num = both[:, :_HF]
    den16 = both[:, _HF:]
    rowh = lax.broadcasted_iota(jnp.int32, (16, _HF), 0)
    colh = lax.broadcasted_iota(jnp.int32, (16, _HF), 1) // _F
    expand = jnp.where(rowh == colh, 1.0, 0.0)
    den128 = jnp.dot(den16, expand, preferred_element_type=jnp.float32)
    h = num / (den128 + 1e-9)
    hf = jnp.where(h > 0, h, jnp.exp(jnp.minimum(h, 0.0)) - 1.0)
    uv = jnp.dot(hf, wp2_ref[...], preferred_element_type=jnp.float32)
    colv = lax.broadcasted_iota(jnp.int32, (_N, 2), 1)
    uv_ref[...] = uv + jnp.where(colv == 0, bp_ref[0], 0.0)


_tc_out = pl.pallas_call(
    _tc_out_body,
    in_specs=[
        pl.BlockSpec(memory_space=pltpu.VMEM),
        pl.BlockSpec(memory_space=pltpu.VMEM),
        pl.BlockSpec(memory_space=pltpu.SMEM),
    ],
    out_shape=jax.ShapeDtypeStruct((_N, 2), jnp.float32),
)


def _sc_pair_body(uv_hbm, dis_hbm, mir_hbm, out_hbm, uvv, dbuf, mbuf, rbuf):
    cid = lax.axis_index("c")
    sid = lax.axis_index("s")
    wid = sid * _NC + cid
    base = wid * _BPT
    pltpu.sync_copy(uv_hbm, uvv)
    pltpu.sync_copy(dis_hbm.at[pl.ds(base, _BPT)], dbuf)
    pltpu.sync_copy(mir_hbm.at[pl.ds(base, _BPT)], mbuf)
    zero16 = jnp.zeros((16,), jnp.int32)
    one16 = jnp.ones((16,), jnp.int32)

    def pair(c0, carry):
        idd = dbuf[pl.ds(c0 * 16, 16)]
        idm = mbuf[pl.ds(c0 * 16, 16)]
        u = plsc.load_gather(uvv, [idd, zero16])
        v = plsc.load_gather(uvv, [idm, one16])
        t = u + v
        rbuf[pl.ds(c0 * 16, 16)] = 1.0 / (1.0 + jnp.exp(-t))
        return carry
    lax.fori_loop(0, _BPT // 16, pair, 0)
    pltpu.sync_copy(rbuf, out_hbm.at[pl.ds(base, _BPT)])


_sc_pair = pl.kernel(
    _sc_pair_body,
    out_type=jax.ShapeDtypeStruct((_B,), jnp.float32),
    mesh=_mesh,
    compiler_params=pltpu.CompilerParams(
        use_tc_tiling_on_sc=False, needs_layout_passes=False),
    scratch_types=[
        pltpu.VMEM((_N, 2), jnp.float32),
        pltpu.VMEM((_BPT,), jnp.int32),
        pltpu.VMEM((_BPT,), jnp.int32),
        pltpu.VMEM((_BPT,), jnp.float32),
    ],
)


def kernel(x, edge_index, diseases, mirnas, W, a_src, a_dst, Wp, bp):
    src2 = edge_index[0].reshape(_E // _CH, _CH)
    dst2 = edge_index[1].reshape(_E // _CH, _CH)
    wflat = jnp.transpose(W, (1, 0, 2)).reshape(_D, _HF)
    asv = a_src.reshape(_HF, 1)
    adv = a_dst.reshape(_HF, 1)
    wp2 = jnp.stack([Wp[:_HF, 0], Wp[_HF:, 0]], axis=1)

    zs, dpad = _tc_proj(x, wflat, asv, adv)
    acc_parts = _sc_edge(src2, dst2, zs, dpad)
    uv = _tc_out(acc_parts, wp2, bp)
    scores = _sc_pair(uv, diseases, mirnas)
    return scores.reshape(_B, 1)
